# baseline (speedup 1.0000x reference)
import dataclasses

import jax
import jax.numpy as jnp
from jax.experimental import pallas as pl
from jax.experimental.pallas import tpu as pltpu
from jax.experimental.pallas import tpu_sc as plsc

_NCLS = 40
_BLK = 4096
_FC = 20480


def _mlp_kernel(x_ref, W1_ref, b1_ref, W2_ref, b2_ref, W3_ref, b3_ref,
                logits_ref):
    x = x_ref[...]
    h = jnp.maximum(
        jnp.dot(x, W1_ref[...], preferred_element_type=jnp.float32)
        + b1_ref[...], 0.0)
    o = jnp.maximum(
        jnp.dot(h, W2_ref[...], preferred_element_type=jnp.float32)
        + b2_ref[...], 0.0)
    logits_ref[...] = (
        jnp.dot(o, W3_ref[...], preferred_element_type=jnp.float32)
        + b3_ref[...])


def _sc_segmean(cu_hbm, invc_hbm, flat_hbm, out_hbm, cu_vbuf,
                invc_vbuf, buf, accbuf, rowbuf, sem):
    c = jax.lax.axis_index("c")
    sj = jax.lax.axis_index("s")
    ncu = cu_hbm.shape[0]
    pltpu.async_copy(cu_hbm, cu_vbuf.at[pl.ds(0, ncu)], sem).wait()
    pltpu.async_copy(invc_hbm, invc_vbuf, sem).wait()

    s = cu_vbuf[pl.ds(sj, 16)][0]
    e = cu_vbuf[pl.ds(sj + 1, 16)][0]
    nrows = e - s
    halfr = (nrows + 1) // 2
    my_s = s + c * halfr
    my_e = jnp.minimum(e, my_s + halfr)
    lo = my_s * _NCLS
    hi = my_e * _NCLS
    nflat = hi - lo
    total = flat_hbm.shape[0]

    zeros = jnp.zeros((16,), jnp.float32)
    lane = jax.lax.iota(jnp.int32, 16)

    def fetch(off):
        pltpu.async_copy(flat_hbm.at[pl.ds(off, _FC)], buf, sem).wait()

    def accum_chunk(acc):
        def body(p, a):
            base = p * 80
            return tuple(a[v] + buf[pl.ds(base + v * 16, 16)]
                         for v in range(5))
        return jax.lax.fori_loop(0, _FC // 80, body, acc)

    def accum_masked(acc, off, lo_abs, hi_abs):
        def body(p, a):
            base = p * 80
            out = []
            for v in range(5):
                pos = off + base + v * 16 + lane
                m = (pos >= lo_abs) & (pos < hi_abs)
                val = buf[pl.ds(base + v * 16, 16)]
                out.append(a[v] + jnp.where(m, val, 0.0))
            return tuple(out)
        return jax.lax.fori_loop(0, _FC // 80, body, acc)

    acc = (zeros, zeros, zeros, zeros, zeros)
    nfull = nflat // _FC

    def main_body(k, a):
        fetch(lo + k * _FC)
        return accum_chunk(a)
    acc = jax.lax.fori_loop(0, nfull, main_body, acc)

    rem = nflat - nfull * _FC
    tail_lo = lo + nfull * _FC
    off = jnp.minimum(tail_lo, total - _FC)

    for v in range(5):
        accbuf[pl.ds(16 * v, 16)] = acc[v]

    @pl.when(rem > 0)
    def _():
        fetch(off)
        a = tuple(accbuf[pl.ds(16 * v, 16)] for v in range(5))
        a = accum_masked(a, off, tail_lo, hi)
        for v in range(5):
            accbuf[pl.ds(16 * v, 16)] = a[v]

    accbuf[pl.ds(80, 16)] = zeros

    inv = invc_vbuf[pl.ds(sj * 16, 16)]
    r0 = accbuf[pl.ds(0, 16)] + plsc.load_gather(accbuf, [lane + 40])
    r1 = accbuf[pl.ds(16, 16)] + plsc.load_gather(accbuf, [lane + 56])
    r2 = accbuf[pl.ds(32, 16)] + plsc.load_gather(accbuf, [lane + 72])
    rowbuf[pl.ds(0, 16)] = r0 * inv
    rowbuf[pl.ds(16, 16)] = r1 * inv
    rowbuf[pl.ds(32, 16)] = r2 * inv
    pltpu.async_copy(rowbuf, out_hbm.at[c, sj], sem).wait()


def kernel(x, cu_seqlens, W1, b1, W2, b2, W3, b3):
    N, D = x.shape
    H = W1.shape[1]
    E = W2.shape[1]
    B = cu_seqlens.shape[0] - 1

    counts = (cu_seqlens[1:] - cu_seqlens[:-1]).astype(jnp.float32)
    inv_rep = jnp.broadcast_to(
        (1.0 / jnp.maximum(counts, 1.0))[:, None], (B, 16)).reshape(-1)

    nb = N // _BLK
    full = lambda shape: pl.BlockSpec(shape, lambda i: (0, 0))

    logits = pl.pallas_call(
        _mlp_kernel,
        grid=(nb,),
        in_specs=[
            pl.BlockSpec((_BLK, D), lambda i: (i, 0)),
            full((D, H)),
            full((1, H)),
            full((H, E)),
            full((1, E)),
            full((E, _NCLS)),
            full((1, _NCLS)),
        ],
        out_specs=pl.BlockSpec((_BLK, _NCLS), lambda i: (i, 0)),
        out_shape=jax.ShapeDtypeStruct((N, _NCLS), jnp.float32),
        compiler_params=pltpu.CompilerParams(
            dimension_semantics=("arbitrary",)),
    )(x, W1, b1.reshape(1, H), W2, b2.reshape(1, E), W3, b3.reshape(1, _NCLS))

    sc_params = pltpu.CompilerParams()
    if "needs_layout_passes" in pltpu.CompilerParams.__dataclass_fields__:
        sc_params = dataclasses.replace(sc_params, needs_layout_passes=False)
    mesh = plsc.VectorSubcoreMesh(core_axis_name="c", subcore_axis_name="s")
    sc_out = pl.kernel(
        _sc_segmean,
        out_type=jax.ShapeDtypeStruct((2, B, 48), jnp.float32),
        mesh=mesh,
        scratch_types=[
            pltpu.VMEM((B + 16,), jnp.int32),
            pltpu.VMEM((B * 16,), jnp.float32),
            pltpu.VMEM((_FC,), jnp.float32),
            pltpu.VMEM((96,), jnp.float32),
            pltpu.VMEM((48,), jnp.float32),
            pltpu.SemaphoreType.DMA,
        ],
        compiler_params=sc_params,
    )(cu_seqlens, inv_rep, logits.reshape(-1))

    global_logits = sc_out[0, :, :_NCLS] + sc_out[1, :, :_NCLS]
    return (global_logits, logits)


# speedup vs baseline: 1.8901x; 1.8901x over previous
import jax
import jax.numpy as jnp
from jax.experimental import pallas as pl
from jax.experimental.pallas import tpu as pltpu

_NCLS = 40
_BLK = 8192


def _fused_kernel(starts_ref, ends_ref, invc_ref, x_ref,
                  W1_ref, b1_ref, W2_ref, b2_ref, W3_ref, b3_ref,
                  sums_ref, logits_ref):
    i = pl.program_id(0)
    nb = pl.num_programs(0)
    B = starts_ref.shape[1]

    x = x_ref[...]
    h = jnp.maximum(
        jnp.dot(x, W1_ref[...], preferred_element_type=jnp.float32)
        + b1_ref[...], 0.0)
    o = jnp.maximum(
        jnp.dot(h, W2_ref[...], preferred_element_type=jnp.float32)
        + b2_ref[...], 0.0)
    logits = (jnp.dot(o, W3_ref[...], preferred_element_type=jnp.float32)
              + b3_ref[...])
    logits_ref[...] = logits

    row = i * _BLK + jax.lax.broadcasted_iota(jnp.int32, (_BLK, B), 0)
    onehot = ((row >= starts_ref[...]) & (row < ends_ref[...])
              ).astype(jnp.float32)
    part = jax.lax.dot_general(
        onehot, logits, (((0,), (0,)), ((), ())),
        preferred_element_type=jnp.float32)

    @pl.when(i == 0)
    def _():
        sums_ref[...] = jnp.zeros_like(sums_ref)

    sums_ref[...] += part

    @pl.when(i == nb - 1)
    def _():
        sums_ref[...] = sums_ref[...] * invc_ref[...]


def kernel(x, cu_seqlens, W1, b1, W2, b2, W3, b3):
    N, D = x.shape
    H = W1.shape[1]
    E = W2.shape[1]
    B = cu_seqlens.shape[0] - 1

    starts = cu_seqlens[:-1].reshape(1, B)
    ends = cu_seqlens[1:].reshape(1, B)
    inv_counts = (1.0 / jnp.maximum(
        (ends - starts).astype(jnp.float32), 1.0)).reshape(B, 1)

    nb = N // _BLK
    grid = (nb,)

    full = lambda shape: pl.BlockSpec(shape, lambda i: (0, 0))

    global_logits, logits = pl.pallas_call(
        _fused_kernel,
        grid=grid,
        in_specs=[
            full((1, B)),
            full((1, B)),
            full((B, 1)),
            pl.BlockSpec((_BLK, D), lambda i: (i, 0)),
            full((D, H)),
            full((1, H)),
            full((H, E)),
            full((1, E)),
            full((E, _NCLS)),
            full((1, _NCLS)),
        ],
        out_specs=[
            full((B, _NCLS)),
            pl.BlockSpec((_BLK, _NCLS), lambda i: (i, 0)),
        ],
        out_shape=[
            jax.ShapeDtypeStruct((B, _NCLS), jnp.float32),
            jax.ShapeDtypeStruct((N, _NCLS), jnp.float32),
        ],
        compiler_params=pltpu.CompilerParams(
            dimension_semantics=("arbitrary",)),
    )(starts, ends, inv_counts, x,
      W1, b1.reshape(1, H),
      W2, b2.reshape(1, E),
      W3, b3.reshape(1, _NCLS))

    return (global_logits, logits)


# speedup vs baseline: 1.9642x; 1.0392x over previous
import jax
import jax.numpy as jnp
from jax.experimental import pallas as pl
from jax.experimental.pallas import tpu as pltpu

_NCLS = 40
_BLK = 4096


def _fused_kernel(starts_ref, ends_ref, invc_ref, x_ref,
                  W1_ref, b1_ref, W2_ref, b2_ref, W3_ref, b3_ref,
                  sums_ref, logits_ref):
    i = pl.program_id(0)
    nb = pl.num_programs(0)
    B = starts_ref.shape[1]

    x = x_ref[...]
    h = jnp.maximum(
        jnp.dot(x, W1_ref[...], preferred_element_type=jnp.float32)
        + b1_ref[...], 0.0)
    o = jnp.maximum(
        jnp.dot(h, W2_ref[...], preferred_element_type=jnp.float32)
        + b2_ref[...], 0.0)
    logits = (jnp.dot(o, W3_ref[...], preferred_element_type=jnp.float32)
              + b3_ref[...])
    logits_ref[...] = logits

    row = i * _BLK + jax.lax.broadcasted_iota(jnp.int32, (_BLK, B), 0)
    onehot = ((row >= starts_ref[...]) & (row < ends_ref[...])
              ).astype(jnp.float32)
    part = jax.lax.dot_general(
        onehot, logits, (((0,), (0,)), ((), ())),
        preferred_element_type=jnp.float32)

    @pl.when(i == 0)
    def _():
        sums_ref[...] = jnp.zeros_like(sums_ref)

    sums_ref[...] += part

    @pl.when(i == nb - 1)
    def _():
        sums_ref[...] = sums_ref[...] * invc_ref[...]


def kernel(x, cu_seqlens, W1, b1, W2, b2, W3, b3):
    N, D = x.shape
    H = W1.shape[1]
    E = W2.shape[1]
    B = cu_seqlens.shape[0] - 1

    starts = cu_seqlens[:-1].reshape(1, B)
    ends = cu_seqlens[1:].reshape(1, B)
    inv_counts = (1.0 / jnp.maximum(
        (ends - starts).astype(jnp.float32), 1.0)).reshape(B, 1)

    nb = N // _BLK
    grid = (nb,)

    full = lambda shape: pl.BlockSpec(shape, lambda i: (0, 0))

    global_logits, logits = pl.pallas_call(
        _fused_kernel,
        grid=grid,
        in_specs=[
            full((1, B)),
            full((1, B)),
            full((B, 1)),
            pl.BlockSpec((_BLK, D), lambda i: (i, 0)),
            full((D, H)),
            full((1, H)),
            full((H, E)),
            full((1, E)),
            full((E, _NCLS)),
            full((1, _NCLS)),
        ],
        out_specs=[
            full((B, _NCLS)),
            pl.BlockSpec((_BLK, _NCLS), lambda i: (i, 0)),
        ],
        out_shape=[
            jax.ShapeDtypeStruct((B, _NCLS), jnp.float32),
            jax.ShapeDtypeStruct((N, _NCLS), jnp.float32),
        ],
        compiler_params=pltpu.CompilerParams(
            dimension_semantics=("arbitrary",)),
    )(starts, ends, inv_counts, x,
      W1, b1.reshape(1, H),
      W2, b2.reshape(1, E),
      W3, b3.reshape(1, _NCLS))

    return (global_logits, logits)
